# baseline (speedup 1.0000x reference)
import functools

import jax
import jax.numpy as jnp
from jax import lax
from jax.experimental import pallas as pl
from jax.experimental.pallas import tpu as pltpu
from jax.experimental.pallas import tpu_sc as plsc

CH = 512


def _gather_sc(idx_hm, table, hist, bsz):
    D = table.shape[1]
    info = plsc.get_sparse_core_info()
    NC, NS = info.num_cores, info.num_subcores
    NW = NC * NS
    blocks = bsz // CH
    n_tasks_total = hist * blocks
    n_tasks = n_tasks_total // NW
    assert n_tasks * NW == n_tasks_total and bsz % CH == 0
    assert n_tasks % 2 == 0 and n_tasks >= 6

    mesh = plsc.VectorSubcoreMesh(core_axis_name="c", subcore_axis_name="s")

    scratch = (
        [pltpu.VMEM((CH,), jnp.int32) for _ in range(2)]
        + [pltpu.VMEM((CH, D), jnp.float32) for _ in range(2)]
        + [pltpu.VMEM((D, CH), jnp.float32) for _ in range(2)]
        + [pltpu.SemaphoreType.DMA for _ in range(6)]
    )

    @functools.partial(
        pl.kernel,
        mesh=mesh,
        out_type=jax.ShapeDtypeStruct((hist, D, bsz), jnp.float32),
        scratch_types=scratch,
        compiler_params=pltpu.CompilerParams(
            use_tc_tiling_on_sc=False, needs_layout_passes=False
        ),
    )
    def k(idx_hbm, table_hbm, out_hbm, idx0, idx1, rows0, rows1, tr0, tr1,
          sem_i0, sem_i1, sem_g0, sem_g1, sem_w0, sem_w1):
        idx_v = (idx0, idx1)
        rows_v = (rows0, rows1)
        trows_v = (tr0, tr1)
        sem_i = (sem_i0, sem_i1)
        sem_g = (sem_g0, sem_g1)
        sem_w = (sem_w0, sem_w1)

        wid = lax.axis_index("s") * NC + lax.axis_index("c")
        g0 = wid * n_tasks

        def copy_i(c, p):
            return pltpu.make_async_copy(
                idx_hbm.at[pl.ds((g0 + c) * CH, CH)], idx_v[p], sem_i[p]
            )

        def copy_g(p):
            return pltpu.make_async_copy(
                table_hbm.at[idx_v[p]], rows_v[p], sem_g[p]
            )

        def copy_w(c, p):
            g = g0 + c
            h = g // blocks
            blk = g % blocks
            return pltpu.make_async_copy(
                trows_v[p], out_hbm.at[h, :, pl.ds(blk * CH, CH)], sem_w[p]
            )

        def transpose(p):
            rows = rows_v[p]
            trows = trows_v[p]

            def body(i, _):
                j0 = i * 16
                ridx = j0 + lax.iota(jnp.int32, 16)
                for d in range(D):
                    col = jnp.full((16,), d, jnp.int32)
                    trows[d, pl.ds(j0, 16)] = plsc.load_gather(
                        rows, [ridx, col]
                    )
                return 0

            lax.fori_loop(0, CH // 16, body, 0)

        def step(c, p, first, last):
            copy_g(p).wait()
            if not last:
                copy_i(c + 2, p).start()
            if not first:
                copy_w(c - 2, p).wait()
            transpose(p)
            copy_w(c, p).start()
            if not last:
                copy_i(c + 2, p).wait()
                copy_g(p).start()

        copy_i(0, 0).start()
        copy_i(1, 1).start()
        copy_i(0, 0).wait()
        copy_g(0).start()
        copy_i(1, 1).wait()
        copy_g(1).start()

        step(0, 0, True, False)
        step(1, 1, True, False)

        def pair(i, _):
            c = 2 * i
            step(c, 0, False, False)
            step(c + 1, 1, False, False)
            return 0

        lax.fori_loop(1, n_tasks // 2 - 1, pair, 0)

        step(n_tasks - 2, 0, False, True)
        step(n_tasks - 1, 1, False, True)
        copy_w(n_tasks - 2, 0).wait()
        copy_w(n_tasks - 1, 1).wait()

    return k(idx_hm, table)


def _mask_tc(term_t):
    def mk(t_ref, o_ref):
        o_ref[...] = t_ref[...] == 0

    return pl.pallas_call(
        mk,
        out_shape=jax.ShapeDtypeStruct(term_t.shape, jnp.bool_),
    )(term_t)


@jax.jit
def kernel(term, table):
    bsz, hist = term.shape
    term_t = term.T
    idx_hm = term_t.reshape(hist * bsz)
    out3 = _gather_sc(idx_hm, table, hist, bsz)
    emb = jnp.transpose(out3, (2, 0, 1))
    mask = _mask_tc(term_t).T
    return emb, mask


# speedup vs baseline: 1.1063x; 1.1063x over previous
import functools

import jax
import jax.numpy as jnp
from jax import lax
from jax.experimental import pallas as pl
from jax.experimental.pallas import tpu as pltpu
from jax.experimental.pallas import tpu_sc as plsc

CH = 512


def _gather_sc(idx_hm, table, hist, bsz):
    D = table.shape[1]
    info = plsc.get_sparse_core_info()
    NC, NS = info.num_cores, info.num_subcores
    NW = NC * NS
    blocks = bsz // CH
    n_tasks_total = hist * blocks
    n_tasks = n_tasks_total // NW
    assert n_tasks * NW == n_tasks_total and bsz % CH == 0
    assert n_tasks % 2 == 0 and n_tasks >= 6

    mesh = plsc.VectorSubcoreMesh(core_axis_name="c", subcore_axis_name="s")

    scratch = (
        [pltpu.VMEM((CH,), jnp.int32) for _ in range(2)]
        + [pltpu.VMEM((CH, D), jnp.float32) for _ in range(2)]
        + [pltpu.VMEM((D, CH), jnp.float32) for _ in range(2)]
        + [pltpu.SemaphoreType.DMA for _ in range(6)]
    )

    @functools.partial(
        pl.kernel,
        mesh=mesh,
        out_type=jax.ShapeDtypeStruct((hist, D, bsz), jnp.float32),
        scratch_types=scratch,
        compiler_params=pltpu.CompilerParams(
            use_tc_tiling_on_sc=False, needs_layout_passes=False
        ),
    )
    def k(idx_hbm, table_hbm, out_hbm, idx0, idx1, rows0, rows1, tr0, tr1,
          sem_i0, sem_i1, sem_g0, sem_g1, sem_w0, sem_w1):
        idx_v = (idx0, idx1)
        rows_v = (rows0, rows1)
        trows_v = (tr0, tr1)
        sem_i = (sem_i0, sem_i1)
        sem_g = (sem_g0, sem_g1)
        sem_w = (sem_w0, sem_w1)

        wid = lax.axis_index("s") * NC + lax.axis_index("c")
        g0 = wid * n_tasks

        def copy_i(c, p):
            return pltpu.make_async_copy(
                idx_hbm.at[pl.ds((g0 + c) * CH, CH)], idx_v[p], sem_i[p]
            )

        def copy_g(p):
            return pltpu.make_async_copy(
                table_hbm.at[idx_v[p]], rows_v[p], sem_g[p]
            )

        def copy_w(c, p):
            g = g0 + c
            h = g // blocks
            blk = g % blocks
            return pltpu.make_async_copy(
                trows_v[p], out_hbm.at[h, :, pl.ds(blk * CH, CH)], sem_w[p]
            )

        def transpose(p):
            rows = rows_v[p]
            trows = trows_v[p]
            d_lo = lax.iota(jnp.int32, 16)
            d_hi = d_lo + 16

            def body(i, _):
                for u in range(8):
                    j = i * 8 + u
                    jv = jnp.full((16,), j, jnp.int32)
                    plsc.store_scatter(
                        trows, [d_lo, jv], rows[j, pl.ds(0, 16)]
                    )
                    plsc.store_scatter(
                        trows, [d_hi, jv], rows[j, pl.ds(16, 16)]
                    )
                return 0

            lax.fori_loop(0, CH // 8, body, 0)

        def step(c, p, first, last):
            copy_g(p).wait()
            if not last:
                copy_i(c + 2, p).start()
            if not first:
                copy_w(c - 2, p).wait()
            transpose(p)
            copy_w(c, p).start()
            if not last:
                copy_i(c + 2, p).wait()
                copy_g(p).start()

        copy_i(0, 0).start()
        copy_i(1, 1).start()
        copy_i(0, 0).wait()
        copy_g(0).start()
        copy_i(1, 1).wait()
        copy_g(1).start()

        step(0, 0, True, False)
        step(1, 1, True, False)

        def pair(i, _):
            c = 2 * i
            step(c, 0, False, False)
            step(c + 1, 1, False, False)
            return 0

        lax.fori_loop(1, n_tasks // 2 - 1, pair, 0)

        step(n_tasks - 2, 0, False, True)
        step(n_tasks - 1, 1, False, True)
        copy_w(n_tasks - 2, 0).wait()
        copy_w(n_tasks - 1, 1).wait()

    return k(idx_hm, table)


def _mask_tc(term_t):
    def mk(t_ref, o_ref):
        o_ref[...] = t_ref[...] == 0

    return pl.pallas_call(
        mk,
        out_shape=jax.ShapeDtypeStruct(term_t.shape, jnp.bool_),
    )(term_t)


@jax.jit
def kernel(term, table):
    bsz, hist = term.shape
    term_t = term.T
    idx_hm = term_t.reshape(hist * bsz)
    out3 = _gather_sc(idx_hm, table, hist, bsz)
    emb = jnp.transpose(out3, (2, 0, 1))
    mask = _mask_tc(term_t).T
    return emb, mask


# speedup vs baseline: 1.5330x; 1.3857x over previous
import functools

import jax
import jax.numpy as jnp
from jax import lax
from jax.experimental import pallas as pl
from jax.experimental.pallas import tpu as pltpu
from jax.experimental.pallas import tpu_sc as plsc

CH = 512
CHP = CH + 1


def _gather_sc(idx_hm, table, hist, bsz):
    D = table.shape[1]
    info = plsc.get_sparse_core_info()
    NC, NS = info.num_cores, info.num_subcores
    NW = NC * NS
    blocks = bsz // CH
    n_tasks_total = hist * blocks
    n_tasks = n_tasks_total // NW
    assert n_tasks * NW == n_tasks_total and bsz % CH == 0
    assert n_tasks % 2 == 0 and n_tasks >= 6

    mesh = plsc.VectorSubcoreMesh(core_axis_name="c", subcore_axis_name="s")

    scratch = (
        [pltpu.VMEM((CH,), jnp.int32) for _ in range(2)]
        + [pltpu.VMEM((CH, D), jnp.float32) for _ in range(2)]
        + [pltpu.VMEM((D, CHP), jnp.float32) for _ in range(2)]
        + [pltpu.SemaphoreType.DMA for _ in range(6)]
    )

    @functools.partial(
        pl.kernel,
        mesh=mesh,
        out_type=jax.ShapeDtypeStruct((hist, D, bsz), jnp.float32),
        scratch_types=scratch,
        compiler_params=pltpu.CompilerParams(
            use_tc_tiling_on_sc=False, needs_layout_passes=False
        ),
    )
    def k(idx_hbm, table_hbm, out_hbm, idx0, idx1, rows0, rows1, tr0, tr1,
          sem_i0, sem_i1, sem_g0, sem_g1, sem_w0, sem_w1):
        idx_v = (idx0, idx1)
        rows_v = (rows0, rows1)
        trows_v = (tr0, tr1)
        sem_i = (sem_i0, sem_i1)
        sem_g = (sem_g0, sem_g1)
        sem_w = (sem_w0, sem_w1)

        wid = lax.axis_index("s") * NC + lax.axis_index("c")
        g0 = wid * n_tasks

        def copy_i(c, p):
            return pltpu.make_async_copy(
                idx_hbm.at[pl.ds((g0 + c) * CH, CH)], idx_v[p], sem_i[p]
            )

        def copy_g(p):
            return pltpu.make_async_copy(
                table_hbm.at[idx_v[p]], rows_v[p], sem_g[p]
            )

        def w_start(c, p):
            g = g0 + c
            h = g // blocks
            blk = g % blocks
            for d in range(D):
                pltpu.make_async_copy(
                    trows_v[p].at[d, pl.ds(0, CH)],
                    out_hbm.at[h, d, pl.ds(blk * CH, CH)],
                    sem_w[p],
                ).start()

        def w_wait(p):
            for d in range(D):
                pltpu.make_async_copy(
                    trows_v[p].at[d, pl.ds(0, CH)],
                    out_hbm.at[0, d, pl.ds(0, CH)],
                    sem_w[p],
                ).wait()

        def transpose(p):
            rows = rows_v[p]
            trows = trows_v[p]
            d_lo = lax.iota(jnp.int32, 16)
            d_hi = d_lo + 16

            def body(i, _):
                for u in range(8):
                    j = i * 8 + u
                    jv = jnp.full((16,), j, jnp.int32)
                    plsc.store_scatter(
                        trows, [d_lo, jv], rows[j, pl.ds(0, 16)]
                    )
                    plsc.store_scatter(
                        trows, [d_hi, jv], rows[j, pl.ds(16, 16)]
                    )
                return 0

            lax.fori_loop(0, CH // 8, body, 0)

        def step(c, p, first, last):
            copy_g(p).wait()
            if not last:
                copy_i(c + 2, p).start()
            if not first:
                w_wait(p)
            transpose(p)
            w_start(c, p)
            if not last:
                copy_i(c + 2, p).wait()
                copy_g(p).start()

        copy_i(0, 0).start()
        copy_i(1, 1).start()
        copy_i(0, 0).wait()
        copy_g(0).start()
        copy_i(1, 1).wait()
        copy_g(1).start()

        step(0, 0, True, False)
        step(1, 1, True, False)

        def pair(i, _):
            c = 2 * i
            step(c, 0, False, False)
            step(c + 1, 1, False, False)
            return 0

        lax.fori_loop(1, n_tasks // 2 - 1, pair, 0)

        step(n_tasks - 2, 0, False, True)
        step(n_tasks - 1, 1, False, True)
        w_wait(0)
        w_wait(1)

    return k(idx_hm, table)


def _mask_tc(term_t):
    def mk(t_ref, o_ref):
        o_ref[...] = t_ref[...] == 0

    return pl.pallas_call(
        mk,
        out_shape=jax.ShapeDtypeStruct(term_t.shape, jnp.bool_),
    )(term_t)


@jax.jit
def kernel(term, table):
    bsz, hist = term.shape
    term_t = term.T
    idx_hm = term_t.reshape(hist * bsz)
    out3 = _gather_sc(idx_hm, table, hist, bsz)
    emb = jnp.transpose(out3, (2, 0, 1))
    mask = _mask_tc(term_t).T
    return emb, mask
